# Initial kernel scaffold; baseline (speedup 1.0000x reference)
import jax
import jax.numpy as jnp
from jax.experimental import pallas as pl


def kernel(x, edge_index, batch, W1, b1, g1, be1, W2, b2, g2, be2, W3, b3, g3, be3, fw1, fb1, fw2, fb2):
    raise NotImplementedError("write your pallas kernel here")


# speedup vs baseline: 28.0222x; 28.0222x over previous
import functools

import jax
import jax.numpy as jnp
from jax import lax
from jax.experimental import pallas as pl
from jax.experimental.pallas import tpu as pltpu
from jax.experimental.pallas import tpu_sc as plsc

N = 50000
E = 800000
G = 128

NPADX = 176
NPAD = N + NPADX
EPAD = 819200
NSTREAM = EPAD // 128
SPT_EDGE = NSTREAM // 32
SPT_FEAT = NSTREAM // 16
KB = 8
ROWS_PT = NPAD // 16
ZCH = 64
BPAD = 65536
BSTREAM = BPAD // 128
GPAD = 384
CROWS_PT = GPAD // 16
GP = G + 8
BN = NPAD // 16

_MESH = plsc.VectorSubcoreMesh(core_axis_name="c", subcore_axis_name="s")
_SC_PARAMS = pltpu.CompilerParams(use_tc_tiling_on_sc=False)


def _zero_fill(zbuf, dw):
    zv = jnp.zeros((16,), jnp.float32)
    for r in range(ZCH):
        for c in range(dw // 16):
            zbuf[r, pl.ds(c * 16, 16)] = zv


def _zero_acc(acc, zbuf, row0):
    def body(k, _):
        pltpu.sync_copy(zbuf, acc.at[pl.ds(row0 + k * ZCH, ZCH)])
        return 0
    lax.fori_loop(0, ROWS_PT // ZCH, body, 0)


@functools.partial(
    pl.kernel,
    out_type=[
        jax.ShapeDtypeStruct((2, NPAD, 16), jnp.float32),
        jax.ShapeDtypeStruct((2, GPAD, 16), jnp.float32),
    ],
    mesh=_MESH,
    compiler_params=_SC_PARAMS,
    scratch_types=[
        pltpu.VMEM((KB, 2, 128), jnp.int32),
        pltpu.VMEM((16, 128), jnp.int32),
        pltpu.VMEM((128, 16), jnp.float32),
        pltpu.VMEM((ZCH, 16), jnp.float32),
        pltpu.VMEM_SHARED((NPAD, 16), jnp.float32),
        pltpu.VMEM_SHARED((GPAD, 16), jnp.float32),
        pltpu.SemaphoreType.DMA,
    ],
)
def _deg_kernel(sd3d, bat2d, degp, cntp, idxb, bblk, ones_v, zbuf, dacc, cacc,
                ssem):
    cid = lax.axis_index("c")
    sid = lax.axis_index("s")
    wid = sid * 2 + cid

    lane = lax.iota(jnp.int32, 16)
    e0 = jnp.where(lane == 0, 1.0, 0.0).astype(jnp.float32)
    for r in range(128):
        ones_v[r] = e0
    _zero_fill(zbuf, 16)
    _zero_acc(dacc, zbuf, sid * ROWS_PT)
    pltpu.sync_copy(zbuf.at[pl.ds(0, CROWS_PT)],
                    cacc.at[pl.ds(sid * CROWS_PT, CROWS_PT)])
    plsc.subcore_barrier()

    def dblk_body(b, _):
        s0 = wid * SPT_EDGE + b * KB
        pltpu.sync_copy(sd3d.at[pl.ds(s0, KB)], idxb)
        ss = [pltpu.async_copy(ones_v, dacc.at[idxb.at[j, 1]], ssem, add=True)
              for j in range(KB)]
        for s in ss:
            s.wait()
        return 0
    lax.fori_loop(0, SPT_EDGE // KB, dblk_body, 0)

    pltpu.sync_copy(bat2d.at[pl.ds(wid * 16, 16)], bblk)
    cs = [pltpu.async_copy(ones_v, cacc.at[bblk.at[j]], ssem, add=True)
          for j in range(16)]
    for c in cs:
        c.wait()

    plsc.subcore_barrier()
    r0 = sid * ROWS_PT
    def out_body(k, _):
        pltpu.sync_copy(dacc.at[pl.ds(r0 + k * ZCH, ZCH)],
                        degp.at[cid, pl.ds(r0 + k * ZCH, ZCH)])
        return 0
    lax.fori_loop(0, ROWS_PT // ZCH, out_body, 0)
    pltpu.sync_copy(cacc.at[pl.ds(sid * CROWS_PT, CROWS_PT)],
                    cntp.at[cid, pl.ds(sid * CROWS_PT, CROWS_PT)])


def _pipe(table, sd3d, base, nblk, kb, idxb, rows, acc, gsem, ssem):
    npair = nblk // 2

    def load_fire_g(p, b):
        pltpu.sync_copy(sd3d.at[pl.ds(base + b * kb, kb)], idxb.at[p])
        for j in range(kb):
            pltpu.async_copy(table.at[idxb.at[p, j, 0]],
                             rows.at[p, pl.ds(j * 128, 128)], gsem)

    def fire_s(p):
        for j in range(kb):
            pltpu.async_copy(rows.at[p, pl.ds(j * 128, 128)],
                             acc.at[idxb.at[p, j, 1]], ssem, add=True)

    def drain(sem, p):
        pltpu.make_async_copy(table.at[pl.ds(0, kb * 128)],
                              rows.at[p], sem).wait()

    load_fire_g(0, 0)

    def pair(i, _):
        b0 = i * 2

        @pl.when(i > 0)
        def _():
            drain(ssem, 1)
        load_fire_g(1, b0 + 1)
        drain(gsem, 0)
        fire_s(0)

        drain(ssem, 0)

        @pl.when(i < npair - 1)
        def _():
            load_fire_g(0, b0 + 2)
        drain(gsem, 1)
        fire_s(1)
        return 0
    lax.fori_loop(0, npair, pair, 0)
    drain(ssem, 1)


def _make_prop_edge(dw, kb):
    @functools.partial(
        pl.kernel,
        out_type=jax.ShapeDtypeStruct((2, NPAD, dw), jnp.float32),
        mesh=_MESH,
        compiler_params=_SC_PARAMS,
        scratch_types=[
            pltpu.VMEM((2, kb, 2, 128), jnp.int32),
            pltpu.VMEM((2, kb * 128, dw), jnp.float32),
            pltpu.VMEM((ZCH, dw), jnp.float32),
            pltpu.VMEM_SHARED((NPAD, dw), jnp.float32),
            pltpu.SemaphoreType.DMA,
            pltpu.SemaphoreType.DMA,
        ],
    )
    def prop(table, sd3d, out, idxb, rows, zbuf, acc, gsem, ssem):
        cid = lax.axis_index("c")
        sid = lax.axis_index("s")
        wid = sid * 2 + cid

        _zero_fill(zbuf, dw)
        _zero_acc(acc, zbuf, sid * ROWS_PT)
        plsc.subcore_barrier()

        _pipe(table, sd3d, wid * SPT_EDGE, SPT_EDGE // kb, kb,
              idxb, rows, acc, gsem, ssem)

        plsc.subcore_barrier()
        r0 = sid * ROWS_PT
        def out_body(k, _):
            pltpu.sync_copy(acc.at[pl.ds(r0 + k * ZCH, ZCH)],
                            out.at[cid, pl.ds(r0 + k * ZCH, ZCH)])
            return 0
        lax.fori_loop(0, ROWS_PT // ZCH, out_body, 0)

    return prop


def _make_prop_feat(dw, kb):
    @functools.partial(
        pl.kernel,
        out_type=jax.ShapeDtypeStruct((2, NPAD, dw), jnp.float32),
        mesh=_MESH,
        compiler_params=_SC_PARAMS,
        scratch_types=[
            pltpu.VMEM((2, kb, 2, 128), jnp.int32),
            pltpu.VMEM((2, kb * 128, dw), jnp.float32),
            pltpu.VMEM((ZCH, dw), jnp.float32),
            pltpu.VMEM_SHARED((NPAD, dw), jnp.float32),
            pltpu.SemaphoreType.DMA,
            pltpu.SemaphoreType.DMA,
        ],
    )
    def prop(tab0, tab1, sd3d, out, idxb, rows, zbuf, acc, gsem, ssem):
        cid = lax.axis_index("c")
        sid = lax.axis_index("s")

        _zero_fill(zbuf, dw)
        _zero_acc(acc, zbuf, sid * ROWS_PT)
        plsc.subcore_barrier()

        for c, tab in ((0, tab0), (1, tab1)):
            @pl.when(cid == c)
            def _():
                _pipe(tab, sd3d, sid * SPT_FEAT, SPT_FEAT // kb, kb,
                      idxb, rows, acc, gsem, ssem)

        plsc.subcore_barrier()
        r0 = sid * ROWS_PT
        def out_body(k, _):
            pltpu.sync_copy(acc.at[pl.ds(r0 + k * ZCH, ZCH)],
                            out.at[cid, pl.ds(r0 + k * ZCH, ZCH)])
            return 0
        lax.fori_loop(0, ROWS_PT // ZCH, out_body, 0)

    return prop


_prop16 = _make_prop_edge(16, 4)
_prop32 = _make_prop_edge(32, 2)
_prop64 = _make_prop_feat(32, 2)


NSTEPS = NPAD // BN


def _stats_update(st_ref, i, z, dout):
    rid = i * BN + lax.broadcasted_iota(jnp.int32, (BN, 1), 0)
    zm = jnp.where(rid < N, z, 0.0)
    upd = jnp.concatenate(
        [jnp.sum(zm, axis=0)[None], jnp.sum(zm * zm, axis=0)[None],
         jnp.zeros((6, dout), jnp.float32)], axis=0)

    @pl.when(i == 0)
    def _():
        st_ref[...] = jnp.zeros((8, dout), jnp.float32)
    st_ref[...] += upd


def _bn_h(z, st_ref, g_ref, be_ref):
    m = st_ref[0] * (1.0 / N)
    v = st_ref[1] * (1.0 / N) - m * m
    a = g_ref[0] * lax.rsqrt(v + 1e-5)
    c = be_ref[0] - m * a
    return jnp.maximum(z * a[None] + c[None], 0.0)


def _tc_b(degp_ref, x_ref, dinv_ref, t1_ref):
    deg = degp_ref[0, :, 0] + degp_ref[1, :, 0] + 1.0
    dv = lax.rsqrt(deg)
    dinv_ref[...] = dv[:, None]
    t1_ref[...] = jnp.concatenate(
        [x_ref[...] * dv[:, None], jnp.zeros((BN, 14), jnp.float32)], axis=1)


def _tc_d1(p_ref, t1_ref, dinv_ref, w_ref, b_ref, z_ref, st_ref):
    i = pl.program_id(0)
    s = (p_ref[0, :, :2] + p_ref[1, :, :2] + t1_ref[:, :2]) * dinv_ref[...]
    z = jnp.dot(s, w_ref[...], preferred_element_type=jnp.float32) + b_ref[...]
    z_ref[...] = z
    _stats_update(st_ref, i, z, 32)


def _tc_d2(z_ref, st_ref, g_ref, be_ref, dinv_ref, t2_ref):
    t2_ref[...] = _bn_h(z_ref[...], st_ref, g_ref, be_ref) * dinv_ref[...]


def _tc_f1(p_ref, t2_ref, dinv_ref, w_ref, b_ref, z_ref, st_ref):
    i = pl.program_id(0)
    s = (p_ref[0] + p_ref[1] + t2_ref[...]) * dinv_ref[...]
    z = jnp.dot(s, w_ref[...], preferred_element_type=jnp.float32) + b_ref[...]
    z_ref[...] = z
    _stats_update(st_ref, i, z, 64)


def _tc_f2(z_ref, st_ref, g_ref, be_ref, dinv_ref, t3a_ref, t3b_ref):
    t3 = _bn_h(z_ref[...], st_ref, g_ref, be_ref) * dinv_ref[...]
    t3a_ref[...] = t3[:, :32]
    t3b_ref[...] = t3[:, 32:]


def _tc_h1(q_ref, t3a_ref, t3b_ref, dinv_ref, w_ref, b_ref, z_ref, st_ref):
    i = pl.program_id(0)
    dv = dinv_ref[...]
    ua = (q_ref[0] + t3a_ref[...]) * dv
    ub = (q_ref[1] + t3b_ref[...]) * dv
    w = w_ref[...]
    z = (jnp.dot(ua, w[:32], preferred_element_type=jnp.float32)
         + jnp.dot(ub, w[32:], preferred_element_type=jnp.float32)
         + b_ref[...])
    z_ref[...] = z
    _stats_update(st_ref, i, z, 128)


def _tc_h2c(z_ref, st_ref, g_ref, be_ref, bat_ref, sums_ref):
    i = pl.program_id(0)
    h = _bn_h(z_ref[...], st_ref, g_ref, be_ref)
    gid = lax.broadcasted_iota(jnp.int32, (GP, BN), 0)
    mask = jnp.where(gid == bat_ref[:, 0][None, :], 1.0, 0.0)

    @pl.when(i == 0)
    def _():
        sums_ref[...] = jnp.zeros((GP, 128), jnp.float32)
    sums_ref[...] += jnp.dot(mask, h, preferred_element_type=jnp.float32)


def _tc_h3(sums_ref, cntp_ref, fw1_ref, fb1_ref, fw2_ref, fb2_ref, out_ref):
    cnt = cntp_ref[0, :G, 0] + cntp_ref[1, :G, 0]
    pooled = sums_ref[:G] / jnp.maximum(cnt, 1.0)[:, None]
    h = jnp.maximum(
        jnp.dot(pooled, fw1_ref[...], preferred_element_type=jnp.float32)
        + fb1_ref[...], 0.0)
    out_ref[...] = jnp.dot(h, fw2_ref[...],
                           preferred_element_type=jnp.float32) + fb2_ref[...]


def _sd(shape):
    return jax.ShapeDtypeStruct(shape, jnp.float32)


def _rows(d):
    return pl.BlockSpec((BN, d), lambda i: (i, 0))


def _parts(d):
    return pl.BlockSpec((2, BN, d), lambda i: (0, i, 0))


def _full(r, c):
    return pl.BlockSpec((r, c), lambda i: (0, 0))


def kernel(x, edge_index, batch, W1, b1, g1, be1, W2, b2, g2, be2,
           W3, b3, g3, be3, fw1, fb1, fw2, fb2):
    pad_n = (N + jnp.arange(EPAD - E, dtype=jnp.int32) % NPADX)
    src2d = jnp.concatenate([edge_index[0], pad_n]).reshape(NSTREAM, 128)
    dst2d = jnp.concatenate([edge_index[1], pad_n]).reshape(NSTREAM, 128)
    sd3d = jnp.stack([src2d, dst2d], axis=1)
    pad_g = (G + jnp.arange(BPAD - N, dtype=jnp.int32) % NPADX)
    bat2d = jnp.concatenate([batch, pad_g]).reshape(BSTREAM, 128)
    batp = jnp.concatenate(
        [batch, jnp.full((NPADX,), G, jnp.int32)])[:, None]
    xp = jnp.concatenate([x, jnp.zeros((NPADX, 2), jnp.float32)], axis=0)

    degp, cntp = _deg_kernel(sd3d, bat2d)

    dinv, t1 = pl.pallas_call(
        _tc_b, grid=(NSTEPS,),
        in_specs=[_parts(16), _rows(2)],
        out_specs=[_rows(1), _rows(16)],
        out_shape=[_sd((NPAD, 1)), _sd((NPAD, 16))],
    )(degp, xp)

    p1 = _prop16(t1, sd3d)

    z1, st1 = pl.pallas_call(
        _tc_d1, grid=(NSTEPS,),
        in_specs=[_parts(16), _rows(16), _rows(1), _full(2, 32), _full(1, 32)],
        out_specs=[_rows(32), _full(8, 32)],
        out_shape=[_sd((NPAD, 32)), _sd((8, 32))],
    )(p1, t1, dinv, W1, b1[None])
    t2 = pl.pallas_call(
        _tc_d2, grid=(NSTEPS,),
        in_specs=[_rows(32), _full(8, 32), _full(1, 32), _full(1, 32),
                  _rows(1)],
        out_specs=_rows(32),
        out_shape=_sd((NPAD, 32)),
    )(z1, st1, g1[None], be1[None], dinv)

    p2 = _prop32(t2, sd3d)

    z2, st2 = pl.pallas_call(
        _tc_f1, grid=(NSTEPS,),
        in_specs=[_parts(32), _rows(32), _rows(1), _full(32, 64),
                  _full(1, 64)],
        out_specs=[_rows(64), _full(8, 64)],
        out_shape=[_sd((NPAD, 64)), _sd((8, 64))],
    )(p2, t2, dinv, W2, b2[None])
    t3a, t3b = pl.pallas_call(
        _tc_f2, grid=(NSTEPS,),
        in_specs=[_rows(64), _full(8, 64), _full(1, 64), _full(1, 64),
                  _rows(1)],
        out_specs=[_rows(32), _rows(32)],
        out_shape=[_sd((NPAD, 32)), _sd((NPAD, 32))],
    )(z2, st2, g2[None], be2[None], dinv)

    q = _prop64(t3a, t3b, sd3d)

    z3, st3 = pl.pallas_call(
        _tc_h1, grid=(NSTEPS,),
        in_specs=[_parts(32), _rows(32), _rows(32), _rows(1),
                  _full(64, 128), _full(1, 128)],
        out_specs=[_rows(128), _full(8, 128)],
        out_shape=[_sd((NPAD, 128)), _sd((8, 128))],
    )(q, t3a, t3b, dinv, W3, b3[None])

    sums = pl.pallas_call(
        _tc_h2c, grid=(NSTEPS,),
        in_specs=[_rows(128), _full(8, 128), _full(1, 128), _full(1, 128),
                  _rows(1)],
        out_specs=_full(GP, 128),
        out_shape=_sd((GP, 128)),
    )(z3, st3, g3[None], be3[None], batp)

    out = pl.pallas_call(_tc_h3, out_shape=_sd((G, 1)))(
        sums, cntp, fw1, fb1[None], fw2, fb2[None])
    return out


# speedup vs baseline: 28.5129x; 1.0175x over previous
import functools

import jax
import jax.numpy as jnp
from jax import lax
from jax.experimental import pallas as pl
from jax.experimental.pallas import tpu as pltpu
from jax.experimental.pallas import tpu_sc as plsc

N = 50000
E = 800000
G = 128

NPADX = 176
NPAD = N + NPADX
EPAD = 819200
NSTREAM = EPAD // 128
SPT_EDGE = NSTREAM // 32
SPT_FEAT = NSTREAM // 16
KB = 8
ROWS_PT = NPAD // 16
ZCH = 64
BPAD = 65536
BSTREAM = BPAD // 128
GPAD = 384
CROWS_PT = GPAD // 16
GP = G + 8
BN = NPAD // 16

_MESH = plsc.VectorSubcoreMesh(core_axis_name="c", subcore_axis_name="s")
_SC_PARAMS = pltpu.CompilerParams(use_tc_tiling_on_sc=False)


def _zero_fill(zbuf, dw):
    zv = jnp.zeros((16,), jnp.float32)
    for r in range(ZCH):
        for c in range(dw // 16):
            zbuf[r, pl.ds(c * 16, 16)] = zv


def _zero_acc(acc, zbuf, row0):
    def body(k, _):
        pltpu.sync_copy(zbuf, acc.at[pl.ds(row0 + k * ZCH, ZCH)])
        return 0
    lax.fori_loop(0, ROWS_PT // ZCH, body, 0)


@functools.partial(
    pl.kernel,
    out_type=[
        jax.ShapeDtypeStruct((2, NPAD, 16), jnp.float32),
        jax.ShapeDtypeStruct((2, GPAD, 16), jnp.float32),
    ],
    mesh=_MESH,
    compiler_params=_SC_PARAMS,
    scratch_types=[
        pltpu.VMEM((KB, 2, 128), jnp.int32),
        pltpu.VMEM((16, 128), jnp.int32),
        pltpu.VMEM((128, 16), jnp.float32),
        pltpu.VMEM((ZCH, 16), jnp.float32),
        pltpu.VMEM_SHARED((NPAD, 16), jnp.float32),
        pltpu.VMEM_SHARED((GPAD, 16), jnp.float32),
        pltpu.SemaphoreType.DMA,
    ],
)
def _deg_kernel(sd3d, bat2d, degp, cntp, idxb, bblk, ones_v, zbuf, dacc, cacc,
                ssem):
    cid = lax.axis_index("c")
    sid = lax.axis_index("s")
    wid = sid * 2 + cid

    lane = lax.iota(jnp.int32, 16)
    e0 = jnp.where(lane == 0, 1.0, 0.0).astype(jnp.float32)
    for r in range(128):
        ones_v[r] = e0
    _zero_fill(zbuf, 16)
    _zero_acc(dacc, zbuf, sid * ROWS_PT)
    pltpu.sync_copy(zbuf.at[pl.ds(0, CROWS_PT)],
                    cacc.at[pl.ds(sid * CROWS_PT, CROWS_PT)])
    plsc.subcore_barrier()

    def dblk_body(b, _):
        s0 = wid * SPT_EDGE + b * KB
        pltpu.sync_copy(sd3d.at[pl.ds(s0, KB)], idxb)
        ss = [pltpu.async_copy(ones_v, dacc.at[idxb.at[j, 1]], ssem, add=True)
              for j in range(KB)]
        for s in ss:
            s.wait()
        return 0
    lax.fori_loop(0, SPT_EDGE // KB, dblk_body, 0)

    pltpu.sync_copy(bat2d.at[pl.ds(wid * 16, 16)], bblk)
    cs = [pltpu.async_copy(ones_v, cacc.at[bblk.at[j]], ssem, add=True)
          for j in range(16)]
    for c in cs:
        c.wait()

    plsc.subcore_barrier()
    r0 = sid * ROWS_PT
    def out_body(k, _):
        pltpu.sync_copy(dacc.at[pl.ds(r0 + k * ZCH, ZCH)],
                        degp.at[cid, pl.ds(r0 + k * ZCH, ZCH)])
        return 0
    lax.fori_loop(0, ROWS_PT // ZCH, out_body, 0)
    pltpu.sync_copy(cacc.at[pl.ds(sid * CROWS_PT, CROWS_PT)],
                    cntp.at[cid, pl.ds(sid * CROWS_PT, CROWS_PT)])


def _pipe(table, sd3d, base, nblk, kb, idxb, rows, acc, gsem, ssem):
    npair = nblk // 2

    def load_fire_g(p, b):
        pltpu.sync_copy(sd3d.at[pl.ds(base + b * kb, kb)], idxb.at[p])
        for j in range(kb):
            pltpu.async_copy(table.at[idxb.at[p, j, 0]],
                             rows.at[p, pl.ds(j * 128, 128)], gsem)

    def fire_s(p):
        for j in range(kb):
            pltpu.async_copy(rows.at[p, pl.ds(j * 128, 128)],
                             acc.at[idxb.at[p, j, 1]], ssem, add=True)

    def drain(sem, p):
        pltpu.make_async_copy(table.at[pl.ds(0, kb * 128)],
                              rows.at[p], sem).wait()

    load_fire_g(0, 0)

    def pair(i, _):
        b0 = i * 2

        @pl.when(i > 0)
        def _():
            drain(ssem, 1)
        load_fire_g(1, b0 + 1)
        drain(gsem, 0)
        fire_s(0)

        drain(ssem, 0)

        @pl.when(i < npair - 1)
        def _():
            load_fire_g(0, b0 + 2)
        drain(gsem, 1)
        fire_s(1)
        return 0
    lax.fori_loop(0, npair, pair, 0)
    drain(ssem, 1)


def _make_prop_edge(dw, kb):
    @functools.partial(
        pl.kernel,
        out_type=jax.ShapeDtypeStruct((2, NPAD, dw), jnp.float32),
        mesh=_MESH,
        compiler_params=_SC_PARAMS,
        scratch_types=[
            pltpu.VMEM((2, kb, 2, 128), jnp.int32),
            pltpu.VMEM((2, kb * 128, dw), jnp.float32),
            pltpu.VMEM((ZCH, dw), jnp.float32),
            pltpu.VMEM_SHARED((NPAD, dw), jnp.float32),
            pltpu.SemaphoreType.DMA,
            pltpu.SemaphoreType.DMA,
        ],
    )
    def prop(table, sd3d, out, idxb, rows, zbuf, acc, gsem, ssem):
        cid = lax.axis_index("c")
        sid = lax.axis_index("s")
        wid = sid * 2 + cid

        _zero_fill(zbuf, dw)
        _zero_acc(acc, zbuf, sid * ROWS_PT)
        plsc.subcore_barrier()

        _pipe(table, sd3d, wid * SPT_EDGE, SPT_EDGE // kb, kb,
              idxb, rows, acc, gsem, ssem)

        plsc.subcore_barrier()
        r0 = sid * ROWS_PT
        def out_body(k, _):
            pltpu.sync_copy(acc.at[pl.ds(r0 + k * ZCH, ZCH)],
                            out.at[cid, pl.ds(r0 + k * ZCH, ZCH)])
            return 0
        lax.fori_loop(0, ROWS_PT // ZCH, out_body, 0)

    return prop


def _make_prop_feat(dw, kb):
    @functools.partial(
        pl.kernel,
        out_type=jax.ShapeDtypeStruct((2, NPAD, dw), jnp.float32),
        mesh=_MESH,
        compiler_params=_SC_PARAMS,
        scratch_types=[
            pltpu.VMEM((2, kb, 2, 128), jnp.int32),
            pltpu.VMEM((2, kb * 128, dw), jnp.float32),
            pltpu.VMEM((ZCH, dw), jnp.float32),
            pltpu.VMEM_SHARED((NPAD, dw), jnp.float32),
            pltpu.SemaphoreType.DMA,
            pltpu.SemaphoreType.DMA,
        ],
    )
    def prop(tab0, tab1, sd3d, out, idxb, rows, zbuf, acc, gsem, ssem):
        cid = lax.axis_index("c")
        sid = lax.axis_index("s")

        _zero_fill(zbuf, dw)
        _zero_acc(acc, zbuf, sid * ROWS_PT)
        plsc.subcore_barrier()

        for c, tab in ((0, tab0), (1, tab1)):
            @pl.when(cid == c)
            def _():
                _pipe(tab, sd3d, sid * SPT_FEAT, SPT_FEAT // kb, kb,
                      idxb, rows, acc, gsem, ssem)

        plsc.subcore_barrier()
        r0 = sid * ROWS_PT
        def out_body(k, _):
            pltpu.sync_copy(acc.at[pl.ds(r0 + k * ZCH, ZCH)],
                            out.at[cid, pl.ds(r0 + k * ZCH, ZCH)])
            return 0
        lax.fori_loop(0, ROWS_PT // ZCH, out_body, 0)

    return prop


_prop16 = _make_prop_edge(16, 4)
_prop32 = _make_prop_edge(32, 2)
_prop64 = _make_prop_feat(32, 2)


NSTEPS = NPAD // BN


def _stats_update(st_ref, i, z, dout):
    rid = i * BN + lax.broadcasted_iota(jnp.int32, (BN, 1), 0)
    zm = jnp.where(rid < N, z, 0.0)
    upd = jnp.concatenate(
        [jnp.sum(zm, axis=0)[None], jnp.sum(zm * zm, axis=0)[None],
         jnp.zeros((6, dout), jnp.float32)], axis=0)

    @pl.when(i == 0)
    def _():
        st_ref[...] = jnp.zeros((8, dout), jnp.float32)
    st_ref[...] += upd


def _bn_h(z, st_ref, g_ref, be_ref):
    m = st_ref[0] * (1.0 / N)
    v = st_ref[1] * (1.0 / N) - m * m
    a = g_ref[0] * lax.rsqrt(v + 1e-5)
    c = be_ref[0] - m * a
    return jnp.maximum(z * a[None] + c[None], 0.0)


def _tc_b(degp_ref, x_ref, dinv_ref, t1_ref):
    deg = degp_ref[0, :, 0] + degp_ref[1, :, 0] + 1.0
    dv = lax.rsqrt(deg)
    dinv_ref[...] = dv[:, None]
    t1_ref[...] = jnp.concatenate(
        [x_ref[...] * dv[:, None], jnp.zeros((BN, 14), jnp.float32)], axis=1)


def _tc_D(p_ref, t1_ref, dinv_ref, w_ref, b_ref, g_ref, be_ref, t2_ref,
          z_scr, st_ref):
    i = pl.program_id(0)

    @pl.when(i < NSTEPS)
    def _():
        s = (p_ref[0, :, :2] + p_ref[1, :, :2] + t1_ref[:, :2]) * dinv_ref[...]
        z = (jnp.dot(s, w_ref[...], preferred_element_type=jnp.float32)
             + b_ref[...])
        z_scr[pl.ds(i * BN, BN)] = z
        _stats_update(st_ref, i, z, 32)

    @pl.when(i >= NSTEPS)
    def _():
        z = z_scr[pl.ds((i - NSTEPS) * BN, BN)]
        t2_ref[...] = _bn_h(z, st_ref, g_ref, be_ref) * dinv_ref[...]


def _tc_F(p_ref, t2_ref, dinv_ref, w_ref, b_ref, g_ref, be_ref,
          t3a_ref, t3b_ref, z_scr, st_ref):
    i = pl.program_id(0)

    @pl.when(i < NSTEPS)
    def _():
        s = (p_ref[0] + p_ref[1] + t2_ref[...]) * dinv_ref[...]
        z = (jnp.dot(s, w_ref[...], preferred_element_type=jnp.float32)
             + b_ref[...])
        z_scr[pl.ds(i * BN, BN)] = z
        _stats_update(st_ref, i, z, 64)

    @pl.when(i >= NSTEPS)
    def _():
        z = z_scr[pl.ds((i - NSTEPS) * BN, BN)]
        t3 = _bn_h(z, st_ref, g_ref, be_ref) * dinv_ref[...]
        t3a_ref[...] = t3[:, :32]
        t3b_ref[...] = t3[:, 32:]


def _tc_H(q_ref, t3a_ref, t3b_ref, dinv_ref, w_ref, b_ref, g_ref, be_ref,
          bat_ref, cntp_ref, fw1_ref, fb1_ref, fw2_ref, fb2_ref, out_ref,
          z_scr, st_ref, sums_ref):
    i = pl.program_id(0)

    @pl.when(i < NSTEPS)
    def _():
        dv = dinv_ref[...]
        ua = (q_ref[0] + t3a_ref[...]) * dv
        ub = (q_ref[1] + t3b_ref[...]) * dv
        w = w_ref[...]
        z = (jnp.dot(ua, w[:32], preferred_element_type=jnp.float32)
             + jnp.dot(ub, w[32:], preferred_element_type=jnp.float32)
             + b_ref[...])
        z_scr[pl.ds(i * BN, BN)] = z
        _stats_update(st_ref, i, z, 128)

    @pl.when(i >= NSTEPS)
    def _():
        h = _bn_h(z_scr[pl.ds((i - NSTEPS) * BN, BN)], st_ref, g_ref, be_ref)
        gid = lax.broadcasted_iota(jnp.int32, (GP, BN), 0)
        mask = jnp.where(gid == bat_ref[:, 0][None, :], 1.0, 0.0)

        @pl.when(i == NSTEPS)
        def _():
            sums_ref[...] = jnp.zeros((GP, 128), jnp.float32)
        sums_ref[...] += jnp.dot(mask, h, preferred_element_type=jnp.float32)

    @pl.when(i == 2 * NSTEPS - 1)
    def _():
        cnt = cntp_ref[0, :G, 0] + cntp_ref[1, :G, 0]
        pooled = sums_ref[:G] / jnp.maximum(cnt, 1.0)[:, None]
        hh = jnp.maximum(
            jnp.dot(pooled, fw1_ref[...], preferred_element_type=jnp.float32)
            + fb1_ref[...], 0.0)
        out_ref[...] = jnp.dot(hh, fw2_ref[...],
                               preferred_element_type=jnp.float32) + fb2_ref[...]


def _sd(shape):
    return jax.ShapeDtypeStruct(shape, jnp.float32)


def _rows(d):
    return pl.BlockSpec((BN, d), lambda i: (i, 0))


def _parts(d):
    return pl.BlockSpec((2, BN, d), lambda i: (0, i, 0))


def _full(r, c):
    return pl.BlockSpec((r, c), lambda i: (0, 0))


def _rows_p0(d):
    return pl.BlockSpec((BN, d), lambda i: (jnp.where(i < NSTEPS, i, 0), 0))


def _rows_p1(d):
    return pl.BlockSpec((BN, d),
                        lambda i: (jnp.where(i < NSTEPS, 0, i - NSTEPS), 0))


def _rows_pb(d):
    return pl.BlockSpec((BN, d), lambda i: (i % NSTEPS, 0))


def _parts_p0(d):
    return pl.BlockSpec((2, BN, d),
                        lambda i: (0, jnp.where(i < NSTEPS, i, 0), 0))


def kernel(x, edge_index, batch, W1, b1, g1, be1, W2, b2, g2, be2,
           W3, b3, g3, be3, fw1, fb1, fw2, fb2):
    pad_n = (N + jnp.arange(EPAD - E, dtype=jnp.int32) % NPADX)
    src2d = jnp.concatenate([edge_index[0], pad_n]).reshape(NSTREAM, 128)
    dst2d = jnp.concatenate([edge_index[1], pad_n]).reshape(NSTREAM, 128)
    sd3d = jnp.stack([src2d, dst2d], axis=1)
    pad_g = (G + jnp.arange(BPAD - N, dtype=jnp.int32) % NPADX)
    bat2d = jnp.concatenate([batch, pad_g]).reshape(BSTREAM, 128)
    batp = jnp.concatenate(
        [batch, jnp.full((NPADX,), G, jnp.int32)])[:, None]
    xp = jnp.concatenate([x, jnp.zeros((NPADX, 2), jnp.float32)], axis=0)

    degp, cntp = _deg_kernel(sd3d, bat2d)

    dinv, t1 = pl.pallas_call(
        _tc_b, grid=(NSTEPS,),
        in_specs=[_parts(16), _rows(2)],
        out_specs=[_rows(1), _rows(16)],
        out_shape=[_sd((NPAD, 1)), _sd((NPAD, 16))],
    )(degp, xp)

    p1 = _prop16(t1, sd3d)

    t2 = pl.pallas_call(
        _tc_D, grid=(2 * NSTEPS,),
        in_specs=[_parts_p0(16), _rows_p0(16), _rows_pb(1), _full(2, 32),
                  _full(1, 32), _full(1, 32), _full(1, 32)],
        out_specs=_rows_pb(32),
        out_shape=_sd((NPAD, 32)),
        scratch_shapes=[pltpu.VMEM((NPAD, 32), jnp.float32),
                        pltpu.VMEM((8, 32), jnp.float32)],
    )(p1, t1, dinv, W1, b1[None], g1[None], be1[None])

    p2 = _prop32(t2, sd3d)

    t3a, t3b = pl.pallas_call(
        _tc_F, grid=(2 * NSTEPS,),
        in_specs=[_parts_p0(32), _rows_p0(32), _rows_pb(1), _full(32, 64),
                  _full(1, 64), _full(1, 64), _full(1, 64)],
        out_specs=[_rows_pb(32), _rows_pb(32)],
        out_shape=[_sd((NPAD, 32)), _sd((NPAD, 32))],
        scratch_shapes=[pltpu.VMEM((NPAD, 64), jnp.float32),
                        pltpu.VMEM((8, 64), jnp.float32)],
    )(p2, t2, dinv, W2, b2[None], g2[None], be2[None])

    q = _prop64(t3a, t3b, sd3d)

    out = pl.pallas_call(
        _tc_H, grid=(2 * NSTEPS,),
        in_specs=[_parts_p0(32), _rows_p0(32), _rows_p0(32), _rows_p0(1),
                  _full(64, 128), _full(1, 128), _full(1, 128),
                  _full(1, 128), _rows_p1(1),
                  pl.BlockSpec((2, GPAD, 16), lambda i: (0, 0, 0)),
                  _full(128, 64), _full(1, 64), _full(64, 1), _full(1, 1)],
        out_specs=_full(G, 1),
        out_shape=_sd((G, 1)),
        scratch_shapes=[pltpu.VMEM((NPAD, 128), jnp.float32),
                        pltpu.VMEM((8, 128), jnp.float32),
                        pltpu.VMEM((GP, 128), jnp.float32)],
    )(q, t3a, t3b, dinv, W3, b3[None], g3[None], be3[None], batp, cntp,
      fw1, fb1[None], fw2, fb2[None])
    return out


# speedup vs baseline: 31.6950x; 1.1116x over previous
import functools

import jax
import jax.numpy as jnp
from jax import lax
from jax.experimental import pallas as pl
from jax.experimental.pallas import tpu as pltpu
from jax.experimental.pallas import tpu_sc as plsc

N = 50000
E = 800000
G = 128

NPADX = 176
NPAD = N + NPADX
EPAD = 819200
NSTREAM = EPAD // 128
SPT_EDGE = NSTREAM // 32
SPT_FEAT = NSTREAM // 16
KB = 8
ROWS_PT = NPAD // 16
ZCH = 64
BPAD = 65536
BSTREAM = BPAD // 128
GPAD = 384
CROWS_PT = GPAD // 16
GP = G + 8
BN = NPAD // 16

_MESH = plsc.VectorSubcoreMesh(core_axis_name="c", subcore_axis_name="s")
_SC_PARAMS = pltpu.CompilerParams(use_tc_tiling_on_sc=False)


def _zero_fill(zbuf, dw):
    zv = jnp.zeros((16,), jnp.float32)
    for r in range(ZCH):
        for c in range(dw // 16):
            zbuf[r, pl.ds(c * 16, 16)] = zv


def _zero_acc(acc, zbuf, row0, zsem):
    cs = [pltpu.async_copy(zbuf, acc.at[pl.ds(row0 + k * ZCH, ZCH)], zsem)
          for k in range(ROWS_PT // ZCH)]
    for c in cs:
        c.wait()


@functools.partial(
    pl.kernel,
    out_type=[
        jax.ShapeDtypeStruct((2, NPAD, 16), jnp.float32),
        jax.ShapeDtypeStruct((2, GPAD, 16), jnp.float32),
    ],
    mesh=_MESH,
    compiler_params=_SC_PARAMS,
    scratch_types=[
        pltpu.VMEM((KB, 2, 128), jnp.int32),
        pltpu.VMEM((16, 128), jnp.int32),
        pltpu.VMEM((128, 16), jnp.float32),
        pltpu.VMEM((ZCH, 16), jnp.float32),
        pltpu.VMEM_SHARED((NPAD, 16), jnp.float32),
        pltpu.VMEM_SHARED((GPAD, 16), jnp.float32),
        pltpu.SemaphoreType.DMA,
    ],
)
def _deg_kernel(sd3d, bat2d, degp, cntp, idxb, bblk, ones_v, zbuf, dacc, cacc,
                ssem):
    cid = lax.axis_index("c")
    sid = lax.axis_index("s")
    wid = sid * 2 + cid

    lane = lax.iota(jnp.int32, 16)
    e0 = jnp.where(lane == 0, 1.0, 0.0).astype(jnp.float32)
    for r in range(128):
        ones_v[r] = e0
    _zero_fill(zbuf, 16)
    _zero_acc(dacc, zbuf, sid * ROWS_PT, ssem)
    pltpu.sync_copy(zbuf.at[pl.ds(0, CROWS_PT)],
                    cacc.at[pl.ds(sid * CROWS_PT, CROWS_PT)])
    plsc.subcore_barrier()

    def dblk_body(b, _):
        s0 = wid * SPT_EDGE + b * KB
        pltpu.sync_copy(sd3d.at[pl.ds(s0, KB)], idxb)
        ss = [pltpu.async_copy(ones_v, dacc.at[idxb.at[j, 1]], ssem, add=True)
              for j in range(KB)]
        for s in ss:
            s.wait()
        return 0
    lax.fori_loop(0, SPT_EDGE // KB, dblk_body, 0)

    pltpu.sync_copy(bat2d.at[pl.ds(wid * 16, 16)], bblk)
    cs = [pltpu.async_copy(ones_v, cacc.at[bblk.at[j]], ssem, add=True)
          for j in range(16)]
    for c in cs:
        c.wait()

    plsc.subcore_barrier()
    r0 = sid * ROWS_PT
    os = [pltpu.async_copy(dacc.at[pl.ds(r0 + k * ZCH, ZCH)],
                           degp.at[cid, pl.ds(r0 + k * ZCH, ZCH)], ssem)
          for k in range(ROWS_PT // ZCH)]
    pltpu.sync_copy(cacc.at[pl.ds(sid * CROWS_PT, CROWS_PT)],
                    cntp.at[cid, pl.ds(sid * CROWS_PT, CROWS_PT)])
    for o in os:
        o.wait()


def _pipe(table, sd3d, base, nblk, kb, idxb, rows, acc, gsem, ssem):
    npair = nblk // 2

    def load_fire_g(p, b):
        pltpu.sync_copy(sd3d.at[pl.ds(base + b * kb, kb)], idxb.at[p])
        for j in range(kb):
            pltpu.async_copy(table.at[idxb.at[p, j, 0]],
                             rows.at[p, pl.ds(j * 128, 128)], gsem)

    def fire_s(p):
        for j in range(kb):
            pltpu.async_copy(rows.at[p, pl.ds(j * 128, 128)],
                             acc.at[idxb.at[p, j, 1]], ssem, add=True)

    def drain(sem, p):
        pltpu.make_async_copy(table.at[pl.ds(0, kb * 128)],
                              rows.at[p], sem).wait()

    load_fire_g(0, 0)

    def pair(i, _):
        b0 = i * 2

        @pl.when(i > 0)
        def _():
            drain(ssem, 1)
        load_fire_g(1, b0 + 1)
        drain(gsem, 0)
        fire_s(0)

        drain(ssem, 0)

        @pl.when(i < npair - 1)
        def _():
            load_fire_g(0, b0 + 2)
        drain(gsem, 1)
        fire_s(1)
        return 0
    lax.fori_loop(0, npair, pair, 0)
    drain(ssem, 1)


def _make_prop_edge(dw, kb):
    @functools.partial(
        pl.kernel,
        out_type=jax.ShapeDtypeStruct((2, NPAD, dw), jnp.float32),
        mesh=_MESH,
        compiler_params=_SC_PARAMS,
        scratch_types=[
            pltpu.VMEM((2, kb, 2, 128), jnp.int32),
            pltpu.VMEM((2, kb * 128, dw), jnp.float32),
            pltpu.VMEM((ZCH, dw), jnp.float32),
            pltpu.VMEM_SHARED((NPAD, dw), jnp.float32),
            pltpu.SemaphoreType.DMA,
            pltpu.SemaphoreType.DMA,
        ],
    )
    def prop(table, sd3d, out, idxb, rows, zbuf, acc, gsem, ssem):
        cid = lax.axis_index("c")
        sid = lax.axis_index("s")
        wid = sid * 2 + cid

        _zero_fill(zbuf, dw)
        _zero_acc(acc, zbuf, sid * ROWS_PT, gsem)
        plsc.subcore_barrier()

        _pipe(table, sd3d, wid * SPT_EDGE, SPT_EDGE // kb, kb,
              idxb, rows, acc, gsem, ssem)

        plsc.subcore_barrier()
        r0 = sid * ROWS_PT
        os = [pltpu.async_copy(acc.at[pl.ds(r0 + k * ZCH, ZCH)],
                               out.at[cid, pl.ds(r0 + k * ZCH, ZCH)], gsem)
              for k in range(ROWS_PT // ZCH)]
        for o in os:
            o.wait()

    return prop


def _make_prop_feat(dw, kb):
    @functools.partial(
        pl.kernel,
        out_type=jax.ShapeDtypeStruct((2, NPAD, dw), jnp.float32),
        mesh=_MESH,
        compiler_params=_SC_PARAMS,
        scratch_types=[
            pltpu.VMEM((2, kb, 2, 128), jnp.int32),
            pltpu.VMEM((2, kb * 128, dw), jnp.float32),
            pltpu.VMEM((ZCH, dw), jnp.float32),
            pltpu.VMEM_SHARED((NPAD, dw), jnp.float32),
            pltpu.SemaphoreType.DMA,
            pltpu.SemaphoreType.DMA,
        ],
    )
    def prop(tab0, tab1, sd3d, out, idxb, rows, zbuf, acc, gsem, ssem):
        cid = lax.axis_index("c")
        sid = lax.axis_index("s")

        _zero_fill(zbuf, dw)
        _zero_acc(acc, zbuf, sid * ROWS_PT, gsem)
        plsc.subcore_barrier()

        for c, tab in ((0, tab0), (1, tab1)):
            @pl.when(cid == c)
            def _():
                _pipe(tab, sd3d, sid * SPT_FEAT, SPT_FEAT // kb, kb,
                      idxb, rows, acc, gsem, ssem)

        plsc.subcore_barrier()
        r0 = sid * ROWS_PT
        os = [pltpu.async_copy(acc.at[pl.ds(r0 + k * ZCH, ZCH)],
                               out.at[cid, pl.ds(r0 + k * ZCH, ZCH)], gsem)
              for k in range(ROWS_PT // ZCH)]
        for o in os:
            o.wait()

    return prop


_prop16 = _make_prop_edge(16, 4)
_prop32 = _make_prop_edge(32, 2)
_prop64 = _make_prop_feat(32, 2)


NSTEPS = NPAD // BN


def _stats_update(st_ref, i, z, dout):
    rid = i * BN + lax.broadcasted_iota(jnp.int32, (BN, 1), 0)
    zm = jnp.where(rid < N, z, 0.0)
    upd = jnp.concatenate(
        [jnp.sum(zm, axis=0)[None], jnp.sum(zm * zm, axis=0)[None],
         jnp.zeros((6, dout), jnp.float32)], axis=0)

    @pl.when(i == 0)
    def _():
        st_ref[...] = jnp.zeros((8, dout), jnp.float32)
    st_ref[...] += upd


def _bn_h(z, st_ref, g_ref, be_ref):
    m = st_ref[0] * (1.0 / N)
    v = st_ref[1] * (1.0 / N) - m * m
    a = g_ref[0] * lax.rsqrt(v + 1e-5)
    c = be_ref[0] - m * a
    return jnp.maximum(z * a[None] + c[None], 0.0)


def _tc_b(degp_ref, x_ref, dinv_ref, t1_ref):
    deg = degp_ref[0, :, 0] + degp_ref[1, :, 0] + 1.0
    dv = lax.rsqrt(deg)
    dinv_ref[...] = dv[:, None]
    t1_ref[...] = jnp.concatenate(
        [x_ref[...] * dv[:, None], jnp.zeros((BN, 14), jnp.float32)], axis=1)


def _tc_D(p_ref, t1_ref, dinv_ref, w_ref, b_ref, g_ref, be_ref, t2_ref,
          z_scr, st_ref):
    i = pl.program_id(0)

    @pl.when(i < NSTEPS)
    def _():
        s = (p_ref[0, :, :2] + p_ref[1, :, :2] + t1_ref[:, :2]) * dinv_ref[...]
        z = (jnp.dot(s, w_ref[...], preferred_element_type=jnp.float32)
             + b_ref[...])
        z_scr[pl.ds(i * BN, BN)] = z
        _stats_update(st_ref, i, z, 32)

    @pl.when(i >= NSTEPS)
    def _():
        z = z_scr[pl.ds((i - NSTEPS) * BN, BN)]
        t2_ref[...] = _bn_h(z, st_ref, g_ref, be_ref) * dinv_ref[...]


def _tc_F(p_ref, t2_ref, dinv_ref, w_ref, b_ref, g_ref, be_ref,
          t3a_ref, t3b_ref, z_scr, st_ref):
    i = pl.program_id(0)

    @pl.when(i < NSTEPS)
    def _():
        s = (p_ref[0] + p_ref[1] + t2_ref[...]) * dinv_ref[...]
        z = (jnp.dot(s, w_ref[...], preferred_element_type=jnp.float32)
             + b_ref[...])
        z_scr[pl.ds(i * BN, BN)] = z
        _stats_update(st_ref, i, z, 64)

    @pl.when(i >= NSTEPS)
    def _():
        z = z_scr[pl.ds((i - NSTEPS) * BN, BN)]
        t3 = _bn_h(z, st_ref, g_ref, be_ref) * dinv_ref[...]
        t3a_ref[...] = t3[:, :32]
        t3b_ref[...] = t3[:, 32:]


def _tc_H(q_ref, t3a_ref, t3b_ref, dinv_ref, w_ref, b_ref, g_ref, be_ref,
          bat_ref, cntp_ref, fw1_ref, fb1_ref, fw2_ref, fb2_ref, out_ref,
          z_scr, st_ref, sums_ref):
    i = pl.program_id(0)

    @pl.when(i < NSTEPS)
    def _():
        dv = dinv_ref[...]
        ua = (q_ref[0] + t3a_ref[...]) * dv
        ub = (q_ref[1] + t3b_ref[...]) * dv
        w = w_ref[...]
        z = (jnp.dot(ua, w[:32], preferred_element_type=jnp.float32)
             + jnp.dot(ub, w[32:], preferred_element_type=jnp.float32)
             + b_ref[...])
        z_scr[pl.ds(i * BN, BN)] = z
        _stats_update(st_ref, i, z, 128)

    @pl.when(i >= NSTEPS)
    def _():
        h = _bn_h(z_scr[pl.ds((i - NSTEPS) * BN, BN)], st_ref, g_ref, be_ref)
        gid = lax.broadcasted_iota(jnp.int32, (GP, BN), 0)
        mask = jnp.where(gid == bat_ref[:, 0][None, :], 1.0, 0.0)

        @pl.when(i == NSTEPS)
        def _():
            sums_ref[...] = jnp.zeros((GP, 128), jnp.float32)
        sums_ref[...] += jnp.dot(mask, h, preferred_element_type=jnp.float32)

    @pl.when(i == 2 * NSTEPS - 1)
    def _():
        cnt = cntp_ref[0, :G, 0] + cntp_ref[1, :G, 0]
        pooled = sums_ref[:G] / jnp.maximum(cnt, 1.0)[:, None]
        hh = jnp.maximum(
            jnp.dot(pooled, fw1_ref[...], preferred_element_type=jnp.float32)
            + fb1_ref[...], 0.0)
        out_ref[...] = jnp.dot(hh, fw2_ref[...],
                               preferred_element_type=jnp.float32) + fb2_ref[...]


def _sd(shape):
    return jax.ShapeDtypeStruct(shape, jnp.float32)


def _rows(d):
    return pl.BlockSpec((BN, d), lambda i: (i, 0))


def _parts(d):
    return pl.BlockSpec((2, BN, d), lambda i: (0, i, 0))


def _full(r, c):
    return pl.BlockSpec((r, c), lambda i: (0, 0))


def _rows_p0(d):
    return pl.BlockSpec((BN, d), lambda i: (jnp.where(i < NSTEPS, i, 0), 0))


def _rows_p1(d):
    return pl.BlockSpec((BN, d),
                        lambda i: (jnp.where(i < NSTEPS, 0, i - NSTEPS), 0))


def _rows_pb(d):
    return pl.BlockSpec((BN, d), lambda i: (i % NSTEPS, 0))


def _parts_p0(d):
    return pl.BlockSpec((2, BN, d),
                        lambda i: (0, jnp.where(i < NSTEPS, i, 0), 0))


def kernel(x, edge_index, batch, W1, b1, g1, be1, W2, b2, g2, be2,
           W3, b3, g3, be3, fw1, fb1, fw2, fb2):
    pad_n = (N + jnp.arange(EPAD - E, dtype=jnp.int32) % NPADX)
    src2d = jnp.concatenate([edge_index[0], pad_n]).reshape(NSTREAM, 128)
    dst2d = jnp.concatenate([edge_index[1], pad_n]).reshape(NSTREAM, 128)
    sd3d = jnp.stack([src2d, dst2d], axis=1)
    pad_g = (G + jnp.arange(BPAD - N, dtype=jnp.int32) % NPADX)
    bat2d = jnp.concatenate([batch, pad_g]).reshape(BSTREAM, 128)
    batp = jnp.concatenate(
        [batch, jnp.full((NPADX,), G, jnp.int32)])[:, None]
    xp = jnp.concatenate([x, jnp.zeros((NPADX, 2), jnp.float32)], axis=0)

    degp, cntp = _deg_kernel(sd3d, bat2d)

    dinv, t1 = pl.pallas_call(
        _tc_b, grid=(NSTEPS,),
        in_specs=[_parts(16), _rows(2)],
        out_specs=[_rows(1), _rows(16)],
        out_shape=[_sd((NPAD, 1)), _sd((NPAD, 16))],
    )(degp, xp)

    p1 = _prop16(t1, sd3d)

    t2 = pl.pallas_call(
        _tc_D, grid=(2 * NSTEPS,),
        in_specs=[_parts_p0(16), _rows_p0(16), _rows_pb(1), _full(2, 32),
                  _full(1, 32), _full(1, 32), _full(1, 32)],
        out_specs=_rows_pb(32),
        out_shape=_sd((NPAD, 32)),
        scratch_shapes=[pltpu.VMEM((NPAD, 32), jnp.float32),
                        pltpu.VMEM((8, 32), jnp.float32)],
    )(p1, t1, dinv, W1, b1[None], g1[None], be1[None])

    p2 = _prop32(t2, sd3d)

    t3a, t3b = pl.pallas_call(
        _tc_F, grid=(2 * NSTEPS,),
        in_specs=[_parts_p0(32), _rows_p0(32), _rows_pb(1), _full(32, 64),
                  _full(1, 64), _full(1, 64), _full(1, 64)],
        out_specs=[_rows_pb(32), _rows_pb(32)],
        out_shape=[_sd((NPAD, 32)), _sd((NPAD, 32))],
        scratch_shapes=[pltpu.VMEM((NPAD, 64), jnp.float32),
                        pltpu.VMEM((8, 64), jnp.float32)],
    )(p2, t2, dinv, W2, b2[None], g2[None], be2[None])

    q = _prop64(t3a, t3b, sd3d)

    out = pl.pallas_call(
        _tc_H, grid=(2 * NSTEPS,),
        in_specs=[_parts_p0(32), _rows_p0(32), _rows_p0(32), _rows_p0(1),
                  _full(64, 128), _full(1, 128), _full(1, 128),
                  _full(1, 128), _rows_p1(1),
                  pl.BlockSpec((2, GPAD, 16), lambda i: (0, 0, 0)),
                  _full(128, 64), _full(1, 64), _full(64, 1), _full(1, 1)],
        out_specs=_full(G, 1),
        out_shape=_sd((G, 1)),
        scratch_shapes=[pltpu.VMEM((NPAD, 128), jnp.float32),
                        pltpu.VMEM((8, 128), jnp.float32),
                        pltpu.VMEM((GP, 128), jnp.float32)],
    )(q, t3a, t3b, dinv, W3, b3[None], g3[None], be3[None], batp, cntp,
      fw1, fb1[None], fw2, fb2[None])
    return out
